# baseline (speedup 1.0000x reference)
import functools

import jax
import jax.numpy as jnp
from jax import lax
from jax.experimental import pallas as pl
from jax.experimental.pallas import tpu as pltpu
from jax.experimental.pallas import tpu_sc as plsc

N = 10000
H = 128
OUT = 16
R = 3
E = 200000

NC = 2
NS = 16
TILES = NC * NS

K = 128
CPT = 50
E_PAD = CPT * TILES * K
PAD = E_PAD - E

WB = 80
NWB = N // WB
WPT = -(-NWB // NS)


def _make_sc_agg(h_dim):
    mesh = plsc.VectorSubcoreMesh(core_axis_name="c", subcore_axis_name="s")

    @functools.partial(
        pl.kernel,
        mesh=mesh,
        out_type=jax.ShapeDtypeStruct((NC, N, h_dim), jnp.float32),
        scratch_types=[
            pltpu.VMEM_SHARED((N + 8, h_dim), jnp.float32),
            pltpu.VMEM((2, K), jnp.int32),
            pltpu.VMEM((2, K), jnp.int32),
            pltpu.VMEM((K, h_dim), jnp.float32),
            pltpu.VMEM((K, h_dim), jnp.float32),
            pltpu.VMEM((WB, h_dim), jnp.float32),
            pltpu.SemaphoreType.DMA,
            pltpu.SemaphoreType.DMA,
        ],
    )
    def agg(t0, t1, t2, eall, zeros, out, acc, eidx0, eidx1, rows0,
            rows1, wbuf, gs0, gs1):
        c = lax.axis_index("c")
        s = lax.axis_index("s")
        wid = c * NS + s

        pltpu.sync_copy(zeros, wbuf)
        for k in range(WPT):
            m = s + NS * k

            @pl.when(m < NWB)
            def _(m=m):
                r0 = pl.multiple_of(m * WB, WB)
                pltpu.sync_copy(wbuf, acc.at[pl.ds(r0, WB)])

        plsc.subcore_barrier()

        base0 = wid * CPT
        for r, tab in enumerate((t0, t1, t2)):
            roff = r * 2 * E_PAD

            def idx_copy(j, eb, roff=roff):
                off = roff + (base0 + j) * K
                pltpu.sync_copy(eall.at[pl.ds(off, K)], eb.at[0])
                pltpu.sync_copy(eall.at[pl.ds(E_PAD + off, K)], eb.at[1])

            def gather(eb, rb, sem, tab=tab):
                pltpu.async_copy(tab.at[eb.at[0]], rb, sem)

            def gather_wait(eb, rb, sem, tab=tab):
                pltpu.make_async_copy(tab.at[eb.at[0]], rb, sem).wait()

            def scatter(eb, rb):
                pltpu.sync_copy(rb, acc.at[eb.at[1]], add=True)

            idx_copy(0, eidx0)
            gather(eidx0, rows0, gs0)
            idx_copy(1, eidx1)
            gather(eidx1, rows1, gs1)

            def step(t, _):
                gather_wait(eidx0, rows0, gs0)
                scatter(eidx0, rows0)
                idx_copy(2 * t + 2, eidx0)
                gather(eidx0, rows0, gs0)
                gather_wait(eidx1, rows1, gs1)
                scatter(eidx1, rows1)
                idx_copy(2 * t + 3, eidx1)
                gather(eidx1, rows1, gs1)
                return 0

            lax.fori_loop(0, CPT // 2 - 1, step, 0)

            gather_wait(eidx0, rows0, gs0)
            scatter(eidx0, rows0)
            gather_wait(eidx1, rows1, gs1)
            scatter(eidx1, rows1)

        plsc.subcore_barrier()

        for k in range(WPT):
            m = s + NS * k

            @pl.when(m < NWB)
            def _(m=m):
                r0 = pl.multiple_of(m * WB, WB)
                pltpu.sync_copy(acc.at[pl.ds(r0, WB)], wbuf)
                pltpu.sync_copy(wbuf, out.at[c, pl.ds(r0, WB)])

    return agg


_sc_agg_h = _make_sc_agg(H)


BN = 400


def _tc_dense_body(p_ref, b_ref, w_ref, o_ref):
    h = jnp.maximum(p_ref[0] + p_ref[1] + b_ref[0], 0.0)
    for r in range(R):
        o_ref[r] = jnp.dot(h, w_ref[r], preferred_element_type=jnp.float32)


def _tc_dense(part, b, w):
    return pl.pallas_call(
        _tc_dense_body,
        grid=(N // BN,),
        in_specs=[
            pl.BlockSpec((NC, BN, H), lambda i: (0, i, 0)),
            pl.BlockSpec((1, H), lambda i: (0, 0)),
            pl.BlockSpec((R, H, H), lambda i: (0, 0, 0)),
        ],
        out_specs=pl.BlockSpec((R, BN, H), lambda i: (0, i, 0)),
        out_shape=jax.ShapeDtypeStruct((R, N, H), jnp.float32),
    )(part, b, w)


def _tc_final_body(p_ref, b_ref, o_ref):
    o_ref[...] = p_ref[0, :, :OUT] + p_ref[1, :, :OUT] + b_ref[0]


def _tc_final(part, b):
    return pl.pallas_call(
        _tc_final_body,
        grid=(N // BN,),
        in_specs=[
            pl.BlockSpec((NC, BN, H), lambda i: (0, i, 0)),
            pl.BlockSpec((1, OUT), lambda i: (0, 0)),
        ],
        out_specs=pl.BlockSpec((BN, OUT), lambda i: (i, 0)),
        out_shape=jax.ShapeDtypeStruct((N, OUT), jnp.float32),
    )(part, b)


@jax.jit
def kernel(embed, b0, w1, b1, w2, b2, edge_index_0, edge_index_1,
           edge_index_2):
    zeros_h = jnp.zeros((WB, H), jnp.float32)
    w2_pad = jnp.zeros((R, H, H), jnp.float32).at[:, :, :OUT].set(w2)
    zpad = jnp.zeros((PAD,), jnp.int32)
    npad = jnp.full((PAD,), N, jnp.int32)
    eall = jnp.concatenate(
        [jnp.concatenate([e[0], zpad, e[1], npad])
         for e in (edge_index_0, edge_index_1, edge_index_2)])

    p0 = _sc_agg_h(embed, embed, embed, eall, zeros_h)
    y = _tc_dense(p0, b0.reshape(1, H), w1)
    p1 = _sc_agg_h(y[0], y[1], y[2], eall, zeros_h)
    z = _tc_dense(p1, b1.reshape(1, H), w2_pad)
    p2 = _sc_agg_h(z[0], z[1], z[2], eall, zeros_h)
    return _tc_final(p2, b2.reshape(1, OUT))


# speedup vs baseline: 1.0003x; 1.0003x over previous
import functools

import jax
import jax.numpy as jnp
from jax import lax
from jax.experimental import pallas as pl
from jax.experimental.pallas import tpu as pltpu
from jax.experimental.pallas import tpu_sc as plsc

N = 10000
H = 128
OUT = 16
R = 3
E = 200000

NC = 2
NS = 16
TILES = NC * NS

K = 128
CPT = 50
E_PAD = CPT * TILES * K
PAD = E_PAD - E

WB = 80
NWB = N // WB
WPT = -(-NWB // NS)


def _make_sc_agg(h_dim):
    mesh = plsc.VectorSubcoreMesh(core_axis_name="c", subcore_axis_name="s")

    @functools.partial(
        pl.kernel,
        mesh=mesh,
        out_type=jax.ShapeDtypeStruct((NC, N, h_dim), jnp.float32),
        scratch_types=[
            pltpu.VMEM_SHARED((N + 8, h_dim), jnp.float32),
            pltpu.VMEM((2, K), jnp.int32),
            pltpu.VMEM((2, K), jnp.int32),
            pltpu.VMEM((K, h_dim), jnp.float32),
            pltpu.VMEM((K, h_dim), jnp.float32),
            pltpu.VMEM((WB, h_dim), jnp.float32),
            pltpu.SemaphoreType.DMA,
            pltpu.SemaphoreType.DMA,
        ],
    )
    def agg(t0, t1, t2, eall, zeros, out, acc, eidx0, eidx1, rows0,
            rows1, wbuf, gs0, gs1):
        c = lax.axis_index("c")
        s = lax.axis_index("s")
        wid = c * NS + s

        pltpu.sync_copy(zeros, wbuf)
        for k in range(WPT):
            m = s + NS * k

            @pl.when(m < NWB)
            def _(m=m):
                r0 = pl.multiple_of(m * WB, WB)
                pltpu.sync_copy(wbuf, acc.at[pl.ds(r0, WB)])

        plsc.subcore_barrier()

        base0 = wid * CPT
        for r, tab in enumerate((t0, t1, t2)):
            roff = r * 2 * E_PAD

            def idx_copy(j, eb, roff=roff):
                off = roff + (base0 + j) * K
                pltpu.sync_copy(eall.at[pl.ds(off, K)], eb.at[0])
                pltpu.sync_copy(eall.at[pl.ds(E_PAD + off, K)], eb.at[1])

            def gather(eb, rb, sem, tab=tab):
                pltpu.async_copy(tab.at[eb.at[0]], rb, sem)

            def gather_wait(eb, rb, sem, tab=tab):
                pltpu.make_async_copy(tab.at[eb.at[0]], rb, sem).wait()

            def scatter(eb, rb):
                pltpu.sync_copy(rb, acc.at[eb.at[1]], add=True)

            idx_copy(0, eidx0)
            gather(eidx0, rows0, gs0)
            idx_copy(1, eidx1)
            gather(eidx1, rows1, gs1)

            def step(t, _):
                gather_wait(eidx0, rows0, gs0)
                scatter(eidx0, rows0)
                idx_copy(2 * t + 2, eidx0)
                gather(eidx0, rows0, gs0)
                gather_wait(eidx1, rows1, gs1)
                scatter(eidx1, rows1)
                idx_copy(2 * t + 3, eidx1)
                gather(eidx1, rows1, gs1)
                return 0

            lax.fori_loop(0, CPT // 2 - 1, step, 0)

            gather_wait(eidx0, rows0, gs0)
            scatter(eidx0, rows0)
            gather_wait(eidx1, rows1, gs1)
            scatter(eidx1, rows1)

        plsc.subcore_barrier()

        for k in range(WPT):
            m = s + NS * k

            @pl.when(m < NWB)
            def _(m=m):
                r0 = pl.multiple_of(m * WB, WB)
                pltpu.sync_copy(acc.at[pl.ds(r0, WB)], wbuf)
                pltpu.sync_copy(wbuf, out.at[c, pl.ds(r0, WB)])

    return agg


_sc_agg_h = _make_sc_agg(H)


BN = 400


def _tc_dense_body(p_ref, b_ref, w_ref, o_ref):
    h = jnp.maximum(p_ref[0] + p_ref[1] + b_ref[0], 0.0)
    for r in range(R):
        o_ref[r] = jnp.dot(h, w_ref[r], preferred_element_type=jnp.float32)


def _tc_dense(part, b, w):
    return pl.pallas_call(
        _tc_dense_body,
        grid=(N // BN,),
        in_specs=[
            pl.BlockSpec((NC, BN, H), lambda i: (0, i, 0)),
            pl.BlockSpec((1, H), lambda i: (0, 0)),
            pl.BlockSpec((R, H, H), lambda i: (0, 0, 0)),
        ],
        out_specs=pl.BlockSpec((R, BN, H), lambda i: (0, i, 0)),
        out_shape=jax.ShapeDtypeStruct((R, N, H), jnp.float32),
    )(part, b, w)


def _tc_final_body(p_ref, b_ref, o_ref):
    o_ref[...] = p_ref[0, :, :OUT] + p_ref[1, :, :OUT] + b_ref[0]


def _tc_final(part, b):
    return pl.pallas_call(
        _tc_final_body,
        grid=(N // BN,),
        in_specs=[
            pl.BlockSpec((NC, BN, H), lambda i: (0, i, 0)),
            pl.BlockSpec((1, OUT), lambda i: (0, 0)),
        ],
        out_specs=pl.BlockSpec((BN, OUT), lambda i: (i, 0)),
        out_shape=jax.ShapeDtypeStruct((N, OUT), jnp.float32),
    )(part, b)


@jax.jit
def kernel(embed, b0, w1, b1, w2, b2, edge_index_0, edge_index_1,
           edge_index_2):
    zeros_h = jnp.zeros((WB, H), jnp.float32)
    w2_pad = jnp.zeros((R, H, H), jnp.float32).at[:, :, :OUT].set(w2)
    zpad = jnp.zeros((PAD,), jnp.int32)
    npad = N + (jnp.arange(PAD, dtype=jnp.int32) % 8)
    eall = jnp.concatenate(
        [jnp.concatenate([e[0], zpad, e[1], npad])
         for e in (edge_index_0, edge_index_1, edge_index_2)])

    p0 = _sc_agg_h(embed, embed, embed, eall, zeros_h)
    y = _tc_dense(p0, b0.reshape(1, H), w1)
    p1 = _sc_agg_h(y[0], y[1], y[2], eall, zeros_h)
    z = _tc_dense(p1, b1.reshape(1, H), w2_pad)
    p2 = _sc_agg_h(z[0], z[1], z[2], eall, zeros_h)
    return _tc_final(p2, b2.reshape(1, OUT))


# speedup vs baseline: 2.8631x; 2.8623x over previous
import functools

import jax
import jax.numpy as jnp
from jax import lax
from jax.experimental import pallas as pl
from jax.experimental.pallas import tpu as pltpu
from jax.experimental.pallas import tpu_sc as plsc

N = 10000
H = 128
OUT = 16
R = 3
E = 200000

NC = 2
NS = 16
TILES = NC * NS

K = 128
CPT = 50
E_PAD = CPT * TILES * K
PAD = E_PAD - E

WB = 80
NWB = N // WB
WPT = -(-NWB // NS)


def _make_sc_agg(h_dim):
    mesh = plsc.VectorSubcoreMesh(core_axis_name="c", subcore_axis_name="s")

    @functools.partial(
        pl.kernel,
        mesh=mesh,
        out_type=jax.ShapeDtypeStruct((NC, N, h_dim), jnp.float32),
        scratch_types=[
            pltpu.VMEM_SHARED((N + 8, h_dim), jnp.float32),
            pltpu.VMEM((2, K), jnp.int32),
            pltpu.VMEM((2, K), jnp.int32),
            pltpu.VMEM((K, h_dim), jnp.float32),
            pltpu.VMEM((K, h_dim), jnp.float32),
            pltpu.VMEM((WB, h_dim), jnp.float32),
            pltpu.SemaphoreType.DMA,
            pltpu.SemaphoreType.DMA,
        ],
    )
    def agg(t0, t1, t2, eall, zeros, out, acc, eidx0, eidx1, rows0,
            rows1, wbuf, gs0, gs1):
        c = lax.axis_index("c")
        s = lax.axis_index("s")
        wid = c * NS + s

        pltpu.sync_copy(zeros, wbuf)
        for k in range(WPT):
            m = s + NS * k

            @pl.when(m < NWB)
            def _(m=m):
                r0 = pl.multiple_of(m * WB, WB)
                pltpu.sync_copy(wbuf, acc.at[pl.ds(r0, WB)])

        plsc.subcore_barrier()

        base0 = wid * CPT
        for r, tab in enumerate((t0, t1, t2)):
            roff = r * 2 * E_PAD

            def idx_copy(j, eb, roff=roff):
                off = roff + (base0 + j) * K
                pltpu.sync_copy(eall.at[pl.ds(off, K)], eb.at[0])
                pltpu.sync_copy(eall.at[pl.ds(E_PAD + off, K)], eb.at[1])

            def gather(eb, rb, sem, tab=tab):
                pltpu.async_copy(tab.at[eb.at[0]], rb, sem)

            def gather_wait(eb, rb, sem, tab=tab):
                pltpu.make_async_copy(tab.at[eb.at[0]], rb, sem).wait()

            def scatter(eb, rb):
                pltpu.sync_copy(rb, acc.at[eb.at[1]], add=True)

            idx_copy(0, eidx0)
            gather(eidx0, rows0, gs0)
            idx_copy(1, eidx1)
            gather(eidx1, rows1, gs1)

            def step(t, _):
                gather_wait(eidx0, rows0, gs0)
                scatter(eidx0, rows0)
                idx_copy(2 * t + 2, eidx0)
                gather(eidx0, rows0, gs0)
                gather_wait(eidx1, rows1, gs1)
                scatter(eidx1, rows1)
                idx_copy(2 * t + 3, eidx1)
                gather(eidx1, rows1, gs1)
                return 0

            lax.fori_loop(0, CPT // 2 - 1, step, 0)

            gather_wait(eidx0, rows0, gs0)
            scatter(eidx0, rows0)
            gather_wait(eidx1, rows1, gs1)
            scatter(eidx1, rows1)

        plsc.subcore_barrier()

        for k in range(WPT):
            m = s + NS * k

            @pl.when(m < NWB)
            def _(m=m):
                r0 = pl.multiple_of(m * WB, WB)
                pltpu.sync_copy(acc.at[pl.ds(r0, WB)], wbuf)
                pltpu.sync_copy(wbuf, out.at[c, pl.ds(r0, WB)])

    return agg


_sc_agg_h = _make_sc_agg(H)


BN = 400


def _tc_dense_body(p_ref, b_ref, w_ref, o_ref):
    h = jnp.maximum(p_ref[0] + p_ref[1] + b_ref[0], 0.0)
    for r in range(R):
        o_ref[r] = jnp.dot(h, w_ref[r], preferred_element_type=jnp.float32)


def _tc_dense(part, b, w):
    return pl.pallas_call(
        _tc_dense_body,
        grid=(N // BN,),
        in_specs=[
            pl.BlockSpec((NC, BN, H), lambda i: (0, i, 0)),
            pl.BlockSpec((1, H), lambda i: (0, 0)),
            pl.BlockSpec((R, H, H), lambda i: (0, 0, 0)),
        ],
        out_specs=pl.BlockSpec((R, BN, H), lambda i: (0, i, 0)),
        out_shape=jax.ShapeDtypeStruct((R, N, H), jnp.float32),
    )(part, b, w)


def _tc_final_body(p_ref, b_ref, o_ref):
    o_ref[...] = p_ref[0, :, :OUT] + p_ref[1, :, :OUT] + b_ref[0]


def _tc_final(part, b):
    return pl.pallas_call(
        _tc_final_body,
        grid=(N // BN,),
        in_specs=[
            pl.BlockSpec((NC, BN, H), lambda i: (0, i, 0)),
            pl.BlockSpec((1, OUT), lambda i: (0, 0)),
        ],
        out_specs=pl.BlockSpec((BN, OUT), lambda i: (i, 0)),
        out_shape=jax.ShapeDtypeStruct((N, OUT), jnp.float32),
    )(part, b)


@jax.jit
def kernel(embed, b0, w1, b1, w2, b2, edge_index_0, edge_index_1,
           edge_index_2):
    zeros_h = jnp.zeros((WB, H), jnp.float32)
    w2_pad = jnp.zeros((R, H, H), jnp.float32).at[:, :, :OUT].set(w2)
    zpad = jnp.arange(PAD, dtype=jnp.int32) % N
    npad = N + (jnp.arange(PAD, dtype=jnp.int32) % 8)
    eall = jnp.concatenate(
        [jnp.concatenate([e[0], zpad, e[1], npad])
         for e in (edge_index_0, edge_index_1, edge_index_2)])

    p0 = _sc_agg_h(embed, embed, embed, eall, zeros_h)
    y = _tc_dense(p0, b0.reshape(1, H), w1)
    p1 = _sc_agg_h(y[0], y[1], y[2], eall, zeros_h)
    z = _tc_dense(p1, b1.reshape(1, H), w2_pad)
    p2 = _sc_agg_h(z[0], z[1], z[2], eall, zeros_h)
    return _tc_final(p2, b2.reshape(1, OUT))
